# baseline (speedup 1.0000x reference)
import functools

import jax
import jax.numpy as jnp
from jax import lax
from jax.experimental import pallas as pl
from jax.experimental.pallas import tpu as pltpu
from jax.experimental.pallas import tpu_sc as plsc


def _sc_gather(table, idx):
    V, D = table.shape
    (B,) = idx.shape
    info = plsc.get_sparse_core_info()
    nw = info.num_cores * info.num_subcores
    b_per_w = B // nw
    mesh = plsc.VectorSubcoreMesh(core_axis_name="c", subcore_axis_name="s")

    @functools.partial(
        pl.kernel,
        mesh=mesh,
        out_type=jax.ShapeDtypeStruct((B, D), jnp.float32),
        scratch_types=[
            pltpu.VMEM((b_per_w,), jnp.int32),
            pltpu.VMEM((b_per_w, D), jnp.float32),
            pltpu.SemaphoreType.DMA,
        ],
    )
    def gather_kernel(table_hbm, idx_hbm, out_hbm, idx_v, rows_v, sem):
        wid = lax.axis_index("s") * info.num_cores + lax.axis_index("c")
        base = wid * b_per_w
        pltpu.sync_copy(idx_hbm.at[pl.ds(base, b_per_w)], idx_v)
        pltpu.async_copy(table_hbm.at[idx_v], rows_v, sem).wait()
        pltpu.sync_copy(rows_v, out_hbm.at[pl.ds(base, b_per_w)])

    return gather_kernel(table, idx)


def _tc_add(X, rows, bb=16):
    B, H, T = X.shape

    def body(x_ref, r_ref, o_ref):
        o_ref[...] = x_ref[...] + r_ref[...][:, :, None]

    return pl.pallas_call(
        body,
        grid=(B // bb,),
        in_specs=[
            pl.BlockSpec((bb, H, T), lambda i: (i, 0, 0)),
            pl.BlockSpec((bb, H), lambda i: (i, 0)),
        ],
        out_specs=pl.BlockSpec((bb, H, T), lambda i: (i, 0, 0)),
        out_shape=jax.ShapeDtypeStruct((B, H, T), jnp.float32),
        compiler_params=pltpu.CompilerParams(
            dimension_semantics=("arbitrary",),
        ),
    )(X, rows)


@jax.jit
def kernel(X, subject_idx, emb):
    rows = _sc_gather(emb, subject_idx.astype(jnp.int32))
    return _tc_add(X, rows)


# speedup vs baseline: 1.0349x; 1.0349x over previous
import functools

import jax
import jax.numpy as jnp
from jax import lax
from jax.experimental import pallas as pl
from jax.experimental.pallas import tpu as pltpu
from jax.experimental.pallas import tpu_sc as plsc


def _sc_gather(table, idx):
    V, D = table.shape
    (B,) = idx.shape
    info = plsc.get_sparse_core_info()
    nw = info.num_cores * info.num_subcores
    b_per_w = B // nw
    mesh = plsc.VectorSubcoreMesh(core_axis_name="c", subcore_axis_name="s")

    @functools.partial(
        pl.kernel,
        mesh=mesh,
        out_type=jax.ShapeDtypeStruct((B, D), jnp.float32),
        scratch_types=[
            pltpu.VMEM((b_per_w,), jnp.int32),
            pltpu.VMEM((b_per_w, D), jnp.float32),
            pltpu.SemaphoreType.DMA,
        ],
    )
    def gather_kernel(table_hbm, idx_hbm, out_hbm, idx_v, rows_v, sem):
        wid = lax.axis_index("s") * info.num_cores + lax.axis_index("c")
        base = wid * b_per_w
        pltpu.sync_copy(idx_hbm.at[pl.ds(base, b_per_w)], idx_v)
        pltpu.async_copy(table_hbm.at[idx_v], rows_v, sem).wait()
        pltpu.sync_copy(rows_v, out_hbm.at[pl.ds(base, b_per_w)])

    return gather_kernel(table, idx)


def _tc_add(X, rows, bb=32, nb=4):
    B, H, T = X.shape
    nc = B // bb

    def body(x_hbm, r_vmem, o_hbm, xbuf, obuf, isem, osem):
        def in_copy(c, s):
            return pltpu.make_async_copy(
                x_hbm.at[pl.ds(c * bb, bb)], xbuf.at[s], isem.at[s])

        def out_copy(c, s):
            return pltpu.make_async_copy(
                obuf.at[s], o_hbm.at[pl.ds(c * bb, bb)], osem.at[s])

        for c in range(min(nb, nc)):
            in_copy(c, c).start()

        def step(c, _):
            s = lax.rem(c, nb)
            in_copy(c, s).wait()

            @pl.when(c >= nb)
            def _():
                out_copy(c - nb, s).wait()

            r = r_vmem[pl.ds(pl.multiple_of(c * bb, bb), bb), :]
            obuf[s] = xbuf[s] + r[:, :, None]

            @pl.when(c + nb < nc)
            def _():
                in_copy(c + nb, s).start()

            out_copy(c, s).start()
            return 0

        lax.fori_loop(0, nc, step, 0)

        for c in range(max(0, nc - nb), nc):
            out_copy(c, c % nb).wait()

    return pl.pallas_call(
        body,
        in_specs=[
            pl.BlockSpec(memory_space=pl.ANY),
            pl.BlockSpec(memory_space=pltpu.VMEM),
        ],
        out_specs=pl.BlockSpec(memory_space=pl.ANY),
        out_shape=jax.ShapeDtypeStruct((B, H, T), jnp.float32),
        scratch_shapes=[
            pltpu.VMEM((nb, bb, H, T), jnp.float32),
            pltpu.VMEM((nb, bb, H, T), jnp.float32),
            pltpu.SemaphoreType.DMA((nb,)),
            pltpu.SemaphoreType.DMA((nb,)),
        ],
    )(X, rows)


@jax.jit
def kernel(X, subject_idx, emb):
    rows = _sc_gather(emb, subject_idx.astype(jnp.int32))
    return _tc_add(X, rows)
